# baseline (speedup 1.0000x reference)
import functools

import jax
import jax.numpy as jnp
from jax import lax
from jax.experimental import pallas as pl
from jax.experimental.pallas import tpu as pltpu
from jax.experimental.pallas import tpu_sc as plsc

B, N, H, K = 4, 2048, 768, 256
NB = 1024


def _v_body(ao_ref, oo_ref, w2a_ref, w2o_ref, va_ref, vo_ref):
    def mm(x_ref, w_ref, o_ref):
        o_ref[0] = lax.dot_general(
            x_ref[0], w_ref[...], (((1,), (0,)), ((), ())),
            preferred_element_type=jnp.float32)
    mm(ao_ref, w2a_ref, va_ref)
    mm(oo_ref, w2o_ref, vo_ref)


def _tc_v(ao, oo, w2a, w2o):
    hspec = pl.BlockSpec((1, NB, H), lambda b, i: (b, i, 0))
    wspec = pl.BlockSpec((H, 3), lambda b, i: (0, 0))
    ospec = pl.BlockSpec((1, NB, 3), lambda b, i: (b, i, 0))
    return pl.pallas_call(
        _v_body,
        grid=(B, N // NB),
        in_specs=[hspec] * 2 + [wspec] * 2,
        out_specs=[ospec] * 2,
        out_shape=[jax.ShapeDtypeStruct((B, N, 3), jnp.float32)] * 2,
    )(ao, oo, w2a, w2o)


def _argmax_body(ha_ref, ho_ref, usrc_ref, w1_ref, b1_ref,
                 aspc_ref, opir_ref, js_ref, u_ref):
    i = pl.program_id(0)
    ha = ha_ref[0]
    ho = ho_ref[0]
    s = lax.dot_general(ha, ho, (((1,), (1,)), ((), ())),
                        preferred_element_type=jnp.float32) / 100.0
    aspc = aspc_ref[0]
    opir = opir_ref[0]
    aspb = jnp.broadcast_to(aspc, (K, K))
    opib = jnp.broadcast_to(opir, (K, K))
    neg = jnp.float32(-3.0e38)
    val = jnp.where(aspb != opib, s, neg)
    m = jnp.max(val, axis=1, keepdims=True)
    cand = jnp.where(val == m, opib, jnp.float32(1e9))
    jmin = jnp.min(cand, axis=1, keepdims=True)
    jstar = jnp.where(m > jnp.float32(-1.0e38), jmin, jnp.float32(0.0))
    js_ref[0] = lax.convert_element_type(jstar, jnp.int32)
    u_in = jnp.where((i % 2) == 1, usrc_ref[0], ha)
    u_ref[0] = lax.dot_general(
        u_in, w1_ref[0], (((1,), (0,)), ((), ())),
        preferred_element_type=jnp.float32) + b1_ref[0]


def _tc_argmax(g_rows, usrc, w1s, b1s, aspc, opir):
    return pl.pallas_call(
        _argmax_body,
        grid=(2 * B,),
        in_specs=[
            pl.BlockSpec((1, K, H), lambda i: (2 * i, 0, 0)),
            pl.BlockSpec((1, K, H), lambda i: (2 * i + 1, 0, 0)),
            pl.BlockSpec((1, K, H), lambda i: (i // 2, 0, 0)),
            pl.BlockSpec((1, H, 3), lambda i: (i % 2, 0, 0)),
            pl.BlockSpec((1, 1, 3), lambda i: (i % 2, 0, 0)),
            pl.BlockSpec((1, K, 1), lambda i: (i, 0, 0)),
            pl.BlockSpec((1, 1, K), lambda i: (i, 0, 0)),
        ],
        out_specs=[
            pl.BlockSpec((1, K, 1), lambda i: (i, 0, 0)),
            pl.BlockSpec((1, K, 3), lambda i: (i, 0, 0)),
        ],
        out_shape=[
            jax.ShapeDtypeStruct((2 * B, K, 1), jnp.int32),
            jax.ShapeDtypeStruct((2 * B, K, 3), jnp.float32),
        ],
    )(g_rows, g_rows, usrc, w1s, b1s, aspc, opir)


def _gather_body(ta_hbm, to_hbm, toa_hbm, idxa_hbm, idxo_hbm, iao_hbm,
                 out_hbm, usrc_hbm, idx_v, idx2_v, idx3_v,
                 rows_v, rows2_v, rows3_v, sem1, sem2, sem3):
    cid = lax.axis_index("c")
    sid = lax.axis_index("s")
    wid = sid * 2 + cid
    b = wid // 8
    s8 = wid % 8
    srcbase = b * (2 * K) + s8 * 64
    off = b * N

    pltpu.sync_copy(idxa_hbm.at[pl.ds(srcbase, 64)], idx_v)
    pltpu.sync_copy(idxo_hbm.at[pl.ds(srcbase, 64)], idx2_v)
    pltpu.sync_copy(iao_hbm.at[b, pl.ds(s8 * 32, 32)], idx3_v)
    for i in range(4):
        idx_v[pl.ds(i * 16, 16)] = idx_v[pl.ds(i * 16, 16)] + off
        idx2_v[pl.ds(i * 16, 16)] = idx2_v[pl.ds(i * 16, 16)] + off
    for i in range(2):
        idx3_v[pl.ds(i * 16, 16)] = idx3_v[pl.ds(i * 16, 16)] + off
    c1 = pltpu.async_copy(ta_hbm.at[idx_v], rows_v, sem1)
    c2 = pltpu.async_copy(to_hbm.at[idx2_v], rows2_v, sem2)
    c3 = pltpu.async_copy(toa_hbm.at[idx3_v], rows3_v, sem3)
    c1.wait()
    pltpu.sync_copy(rows_v, out_hbm.at[pl.ds(b * (4 * K) + s8 * 64, 64)])
    c2.wait()
    pltpu.sync_copy(rows2_v,
                    out_hbm.at[pl.ds(b * (4 * K) + 2 * K + s8 * 64, 64)])
    c3.wait()
    pltpu.sync_copy(rows3_v, usrc_hbm.at[pl.ds(b * K + s8 * 32, 32)])


def _sc_gather(ta_flat, to_flat, toa_flat, idx_a, idx_o, iao):
    mesh = plsc.VectorSubcoreMesh(core_axis_name="c", subcore_axis_name="s")
    fn = pl.kernel(
        _gather_body,
        out_type=[jax.ShapeDtypeStruct((4 * B * K, H), jnp.float32),
                  jax.ShapeDtypeStruct((B * K, H), jnp.float32)],
        mesh=mesh,
        scratch_types=[
            pltpu.VMEM((64,), jnp.int32),
            pltpu.VMEM((64,), jnp.int32),
            pltpu.VMEM((32,), jnp.int32),
            pltpu.VMEM((64, H), jnp.float32),
            pltpu.VMEM((64, H), jnp.float32),
            pltpu.VMEM((32, H), jnp.float32),
            pltpu.SemaphoreType.DMA,
            pltpu.SemaphoreType.DMA,
            pltpu.SemaphoreType.DMA,
        ],
    )
    return fn(ta_flat, to_flat, toa_flat, idx_a, idx_o, iao)


def _combine_body(us, va, vo, js, iaa, iao, zz, out_rm, out_t,
                  ua_v, va_v, uo_v, vo_v, fa_v, fo_v, frm_v,
                  ia_v, ja_v, io_v, jo_v):
    cid = lax.axis_index("c")
    sid = lax.axis_index("s")
    wid = sid * 2 + cid
    lane = jnp.arange(16, dtype=jnp.int32)

    @pl.when(wid < B)
    def _():
        b = wid
        pltpu.sync_copy(zz, fa_v)
        pltpu.sync_copy(zz, fo_v)
        pltpu.sync_copy(us.at[2 * b], ua_v)
        pltpu.sync_copy(va.at[b], va_v)
        pltpu.sync_copy(us.at[2 * b + 1], uo_v)
        pltpu.sync_copy(vo.at[b], vo_v)
        pltpu.sync_copy(iaa.at[b], ia_v)
        pltpu.sync_copy(js.at[2 * b], ja_v)
        pltpu.sync_copy(iao.at[b], io_v)
        pltpu.sync_copy(js.at[2 * b + 1], jo_v)

        def mk(i_ref, j_ref, u_ref, v_ref, f_ref):
            def cb(t, _):
                ii = i_ref[pl.ds(t * 16, 16)]
                jj = j_ref[pl.ds(t * 16, 16)]
                pv = lane + t * 16
                for c in range(3):
                    u = plsc.load_gather(u_ref, [pv * 3 + c])
                    v = plsc.load_gather(v_ref, [jj * 3 + c])
                    plsc.store_scatter(f_ref, [ii + c * N], u + v)
                return 0
            lax.fori_loop(0, K // 16, cb, 0)

        mk(ia_v, ja_v, ua_v, va_v, fa_v)
        mk(io_v, jo_v, uo_v, vo_v, fo_v)

        def ab(i, _):
            sl = pl.ds(i * 16, 16)
            v = (fa_v[sl] + fo_v[sl]) * 0.5
            fa_v[sl] = v
            base = lane + i * 16
            c = base // N
            row = base - c * N
            plsc.store_scatter(frm_v, [row * 3 + c], v)
            return 0
        lax.fori_loop(0, (3 * N) // 16, ab, 0)
        pltpu.sync_copy(frm_v, out_rm.at[b])
        pltpu.sync_copy(fa_v, out_t.at[b])


def _sc_combine(us, va, vo, js, iaa, iao, zz):
    mesh = plsc.VectorSubcoreMesh(core_axis_name="c", subcore_axis_name="s")
    fn = pl.kernel(
        _combine_body,
        out_type=[jax.ShapeDtypeStruct((B, 3 * N), jnp.float32),
                  jax.ShapeDtypeStruct((B, 3 * N), jnp.float32)],
        mesh=mesh,
        compiler_params=pltpu.CompilerParams(needs_layout_passes=False),
        scratch_types=[
            pltpu.VMEM((3 * K,), jnp.float32),
            pltpu.VMEM((3 * N,), jnp.float32),
            pltpu.VMEM((3 * K,), jnp.float32),
            pltpu.VMEM((3 * N,), jnp.float32),
            pltpu.VMEM((3 * N,), jnp.float32),
            pltpu.VMEM((3 * N,), jnp.float32),
            pltpu.VMEM((3 * N,), jnp.float32),
            pltpu.VMEM((K,), jnp.int32),
            pltpu.VMEM((K,), jnp.int32),
            pltpu.VMEM((K,), jnp.int32),
            pltpu.VMEM((K,), jnp.int32),
        ],
    )
    return fn(us, va, vo, js, iaa, iao, zz)


def _loss_body(fl_ref, lab_ref, pred_ref, loss_ref):
    b = pl.program_id(0)
    f = fl_ref[0]
    lab = lab_ref[0]
    f0, f1, f2 = f[0:1, :], f[1:2, :], f[2:3, :]
    absum = jnp.abs(f0) + jnp.abs(f1) + jnp.abs(f2)
    valid = (absum > 0).astype(jnp.float32)
    mx = jnp.maximum(jnp.maximum(f0, f1), f2)
    se = jnp.exp(f0 - mx) + jnp.exp(f1 - mx) + jnp.exp(f2 - mx)
    lse = jnp.log(se) + mx
    flab = jnp.where(lab == 0, f0, jnp.where(lab == 1, f1, f2))
    nll = lse - flab
    wl = jnp.where(lab == 0, jnp.float32(1.0),
                   jnp.where(lab == 1, jnp.float32(2.0), jnp.float32(4.0)))
    wl = wl * valid
    num = jnp.sum(nll * wl, axis=(0, 1), keepdims=True)
    den = jnp.maximum(jnp.sum(wl, axis=(0, 1), keepdims=True),
                      jnp.float32(1e-6))
    lossb = num / den
    idx = jnp.zeros_like(lab)
    best = f0
    idx = jnp.where(f1 > best, 1, idx)
    best = jnp.maximum(best, f1)
    idx = jnp.where(f2 > best, 2, idx)
    pred_ref[0] = jnp.where(valid > 0, idx, -1)

    @pl.when(b == 0)
    def _():
        loss_ref[...] = lossb

    @pl.when(b > 0)
    def _():
        loss_ref[...] = loss_ref[...] + lossb


def _tc_loss(fl_t, lab3):
    return pl.pallas_call(
        _loss_body,
        grid=(B,),
        in_specs=[
            pl.BlockSpec((1, 3, N), lambda b: (b, 0, 0)),
            pl.BlockSpec((1, 1, N), lambda b: (b, 0, 0)),
        ],
        out_specs=[
            pl.BlockSpec((1, 1, N), lambda b: (b, 0, 0)),
            pl.BlockSpec((1, 1), lambda b: (0, 0)),
        ],
        out_shape=[
            jax.ShapeDtypeStruct((B, 1, N), jnp.int32),
            jax.ShapeDtypeStruct((1, 1), jnp.float32),
        ],
    )(fl_t, lab3)


def kernel(A2O_aspect_hidden_states, A2O_opinion_hidden_states,
           O2A_aspect_hidden_states, O2A_opinion_hidden_states,
           W_A2O, b_A2O, W_O2A, b_O2A,
           asp_idx_a2o, opi_idx_a2o, asp_idx_o2a, opi_idx_o2a,
           sentiment_labels):
    aa = A2O_aspect_hidden_states.astype(jnp.float32)
    ao = A2O_opinion_hidden_states.astype(jnp.float32)
    oa = O2A_aspect_hidden_states.astype(jnp.float32)
    oo = O2A_opinion_hidden_states.astype(jnp.float32)
    ia_a = asp_idx_a2o.astype(jnp.int32)
    ja_a = opi_idx_a2o.astype(jnp.int32)
    ia_o = asp_idx_o2a.astype(jnp.int32)
    ja_o = opi_idx_o2a.astype(jnp.int32)

    idx_a = jnp.stack([ia_a, ja_a], axis=1).reshape(2 * B * K)
    idx_o = jnp.stack([ia_o, ja_o], axis=1).reshape(2 * B * K)
    g_rows, usrc = _sc_gather(aa.reshape(B * N, H), oo.reshape(B * N, H),
                              oa.reshape(B * N, H), idx_a, idx_o, ia_o)
    g_rows = g_rows.reshape(4 * B, K, H)
    usrc = usrc.reshape(B, K, H)

    va, vo = _tc_v(ao, oo, W_A2O[H:], W_O2A[H:])

    aspc = jnp.stack([ia_a, ia_o], axis=1).reshape(2 * B, K, 1)
    opir = jnp.stack([ja_a, ja_o], axis=1).reshape(2 * B, 1, K)
    w1s = jnp.stack([W_A2O[:H], W_O2A[:H]]).astype(jnp.float32)
    b1s = jnp.stack([b_A2O.reshape(1, 3), b_O2A.reshape(1, 3)]).astype(jnp.float32)
    js, us = _tc_argmax(g_rows, usrc, w1s, b1s, aspc.astype(jnp.float32),
                        opir.astype(jnp.float32))
    js = js.reshape(2 * B, K)

    fl_rm, fl_t = _sc_combine(us.reshape(2 * B, 3 * K),
                              va.reshape(B, 3 * N), vo.reshape(B, 3 * N),
                              js, ia_a, ia_o,
                              jnp.zeros((3 * N,), jnp.float32))
    fl = fl_rm.reshape(B, N, 3)

    lab3 = sentiment_labels.astype(jnp.int32).reshape(B, 1, N)
    pred, loss = _tc_loss(fl_t.reshape(B, 3, N), lab3)
    return fl, pred.reshape(B, N), loss.reshape(())


# speedup vs baseline: 1.0161x; 1.0161x over previous
import jax
import jax.numpy as jnp
from jax import lax
from jax.experimental import pallas as pl
from jax.experimental.pallas import tpu as pltpu
from jax.experimental.pallas import tpu_sc as plsc

B, N, H, K = 4, 2048, 768, 256
NB = 1024


def _v_body(ao_ref, oo_ref, w2a_ref, w2o_ref, va_ref, vo_ref):
    def mm(x_ref, w_ref, o_ref):
        o_ref[0] = lax.dot_general(
            x_ref[0], w_ref[...], (((1,), (0,)), ((), ())),
            preferred_element_type=jnp.float32)
    mm(ao_ref, w2a_ref, va_ref)
    mm(oo_ref, w2o_ref, vo_ref)


def _tc_v(ao, oo, w2a, w2o):
    hspec = pl.BlockSpec((1, NB, H), lambda b, i: (b, i, 0))
    wspec = pl.BlockSpec((H, 3), lambda b, i: (0, 0))
    ospec = pl.BlockSpec((1, NB, 3), lambda b, i: (b, i, 0))
    return pl.pallas_call(
        _v_body,
        grid=(B, N // NB),
        in_specs=[hspec] * 2 + [wspec] * 2,
        out_specs=[ospec] * 2,
        out_shape=[jax.ShapeDtypeStruct((B, N, 3), jnp.float32)] * 2,
    )(ao, oo, w2a, w2o)


def _argmax_body(ha_ref, ho_ref, usrc_ref, w1_ref, b1_ref,
                 aspc_ref, opir_ref, js_ref, u_ref):
    i = pl.program_id(0)
    ha = ha_ref[0]
    ho = ho_ref[0]
    s = lax.dot_general(ha, ho, (((1,), (1,)), ((), ())),
                        preferred_element_type=jnp.float32) / 100.0
    aspc = aspc_ref[0]
    opir = opir_ref[0]
    aspb = jnp.broadcast_to(aspc, (K, K))
    opib = jnp.broadcast_to(opir, (K, K))
    neg = jnp.float32(-3.0e38)
    val = jnp.where(aspb != opib, s, neg)
    m = jnp.max(val, axis=1, keepdims=True)
    cand = jnp.where(val == m, opib, jnp.float32(1e9))
    jmin = jnp.min(cand, axis=1, keepdims=True)
    jstar = jnp.where(m > jnp.float32(-1.0e38), jmin, jnp.float32(0.0))
    js_ref[0] = lax.convert_element_type(jstar, jnp.int32)
    u_in = jnp.where((i % 2) == 1, usrc_ref[0], ha)
    u_ref[0] = lax.dot_general(
        u_in, w1_ref[0], (((1,), (0,)), ((), ())),
        preferred_element_type=jnp.float32) + b1_ref[0]


def _tc_argmax(g_rows, usrc, w1s, b1s, aspc, opir):
    return pl.pallas_call(
        _argmax_body,
        grid=(2 * B,),
        in_specs=[
            pl.BlockSpec((1, K, H), lambda i: (2 * i, 0, 0)),
            pl.BlockSpec((1, K, H), lambda i: (2 * i + 1, 0, 0)),
            pl.BlockSpec((1, K, H), lambda i: (i // 2, 0, 0)),
            pl.BlockSpec((1, H, 3), lambda i: (i % 2, 0, 0)),
            pl.BlockSpec((1, 1, 3), lambda i: (i % 2, 0, 0)),
            pl.BlockSpec((1, K, 1), lambda i: (i, 0, 0)),
            pl.BlockSpec((1, 1, K), lambda i: (i, 0, 0)),
        ],
        out_specs=[
            pl.BlockSpec((1, K, 1), lambda i: (i, 0, 0)),
            pl.BlockSpec((1, K, 3), lambda i: (i, 0, 0)),
        ],
        out_shape=[
            jax.ShapeDtypeStruct((2 * B, K, 1), jnp.int32),
            jax.ShapeDtypeStruct((2 * B, K, 3), jnp.float32),
        ],
    )(g_rows, g_rows, usrc, w1s, b1s, aspc, opir)


def _gather_body(ta_hbm, to_hbm, toa_hbm, idxa_hbm, idxo_hbm, iao_hbm,
                 out_hbm, usrc_hbm, idx_v, idx2_v, idx3_v,
                 rows_v, rows2_v, rows3_v, sem1, sem2, sem3):
    cid = lax.axis_index("c")
    sid = lax.axis_index("s")
    wid = sid * 2 + cid
    b = wid // 8
    s8 = wid % 8
    srcbase = b * (2 * K) + s8 * 64
    off = b * N

    pltpu.sync_copy(idxa_hbm.at[pl.ds(srcbase, 64)], idx_v)
    pltpu.sync_copy(idxo_hbm.at[pl.ds(srcbase, 64)], idx2_v)
    pltpu.sync_copy(iao_hbm.at[b, pl.ds(s8 * 32, 32)], idx3_v)
    for i in range(4):
        idx_v[pl.ds(i * 16, 16)] = idx_v[pl.ds(i * 16, 16)] + off
        idx2_v[pl.ds(i * 16, 16)] = idx2_v[pl.ds(i * 16, 16)] + off
    for i in range(2):
        idx3_v[pl.ds(i * 16, 16)] = idx3_v[pl.ds(i * 16, 16)] + off
    c1 = pltpu.async_copy(ta_hbm.at[idx_v], rows_v, sem1)
    c2 = pltpu.async_copy(to_hbm.at[idx2_v], rows2_v, sem2)
    c3 = pltpu.async_copy(toa_hbm.at[idx3_v], rows3_v, sem3)
    c1.wait()
    pltpu.sync_copy(rows_v, out_hbm.at[pl.ds(b * (4 * K) + s8 * 64, 64)])
    c2.wait()
    pltpu.sync_copy(rows2_v,
                    out_hbm.at[pl.ds(b * (4 * K) + 2 * K + s8 * 64, 64)])
    c3.wait()
    pltpu.sync_copy(rows3_v, usrc_hbm.at[pl.ds(b * K + s8 * 32, 32)])


def _sc_gather(ta_flat, to_flat, toa_flat, idx_a, idx_o, iao):
    mesh = plsc.VectorSubcoreMesh(core_axis_name="c", subcore_axis_name="s")
    fn = pl.kernel(
        _gather_body,
        out_type=[jax.ShapeDtypeStruct((4 * B * K, H), jnp.float32),
                  jax.ShapeDtypeStruct((B * K, H), jnp.float32)],
        mesh=mesh,
        scratch_types=[
            pltpu.VMEM((64,), jnp.int32),
            pltpu.VMEM((64,), jnp.int32),
            pltpu.VMEM((32,), jnp.int32),
            pltpu.VMEM((64, H), jnp.float32),
            pltpu.VMEM((64, H), jnp.float32),
            pltpu.VMEM((32, H), jnp.float32),
            pltpu.SemaphoreType.DMA,
            pltpu.SemaphoreType.DMA,
            pltpu.SemaphoreType.DMA,
        ],
    )
    return fn(ta_flat, to_flat, toa_flat, idx_a, idx_o, iao)


def _combine_body(us, va, vo, js, iaa, iao, zz, out_rm, out_t,
                  ua_v, va_v, uo_v, vo_v, fa_v, fo_v, frm_v,
                  ia_v, ja_v, io_v, jo_v):
    cid = lax.axis_index("c")
    sid = lax.axis_index("s")
    wid = sid * 2 + cid
    lane = jnp.arange(16, dtype=jnp.int32)

    @pl.when(wid < B)
    def _():
        b = wid
        pltpu.sync_copy(zz, fa_v)
        pltpu.sync_copy(zz, fo_v)
        pltpu.sync_copy(us.at[2 * b], ua_v)
        pltpu.sync_copy(va.at[b], va_v)
        pltpu.sync_copy(us.at[2 * b + 1], uo_v)
        pltpu.sync_copy(vo.at[b], vo_v)
        pltpu.sync_copy(iaa.at[b], ia_v)
        pltpu.sync_copy(js.at[2 * b], ja_v)
        pltpu.sync_copy(iao.at[b], io_v)
        pltpu.sync_copy(js.at[2 * b + 1], jo_v)

        def mk(i_ref, j_ref, u_ref, v_ref, f_ref):
            def cb(t, _):
                ii = i_ref[pl.ds(t * 16, 16)]
                jj = j_ref[pl.ds(t * 16, 16)]
                pv = lane + t * 16
                for c in range(3):
                    u = plsc.load_gather(u_ref, [pv * 3 + c])
                    v = plsc.load_gather(v_ref, [jj * 3 + c])
                    plsc.store_scatter(f_ref, [ii + c * N], u + v)
                return 0
            lax.fori_loop(0, K // 16, cb, 0)

        mk(ia_v, ja_v, ua_v, va_v, fa_v)
        mk(io_v, jo_v, uo_v, vo_v, fo_v)

        def ab(i, _):
            sl = pl.ds(i * 16, 16)
            v = (fa_v[sl] + fo_v[sl]) * 0.5
            fa_v[sl] = v
            base = lane + i * 16
            c = base // N
            row = base - c * N
            plsc.store_scatter(frm_v, [row * 3 + c], v)
            return 0
        lax.fori_loop(0, (3 * N) // 16, ab, 0)
        pltpu.sync_copy(frm_v, out_rm.at[b])
        pltpu.sync_copy(fa_v, out_t.at[b])


def _sc_combine(us, va, vo, js, iaa, iao, zz):
    mesh = plsc.VectorSubcoreMesh(core_axis_name="c", subcore_axis_name="s")
    fn = pl.kernel(
        _combine_body,
        out_type=[jax.ShapeDtypeStruct((B, 3 * N), jnp.float32),
                  jax.ShapeDtypeStruct((B, 3 * N), jnp.float32)],
        mesh=mesh,
        compiler_params=pltpu.CompilerParams(needs_layout_passes=False),
        scratch_types=[
            pltpu.VMEM((3 * K,), jnp.float32),
            pltpu.VMEM((3 * N,), jnp.float32),
            pltpu.VMEM((3 * K,), jnp.float32),
            pltpu.VMEM((3 * N,), jnp.float32),
            pltpu.VMEM((3 * N,), jnp.float32),
            pltpu.VMEM((3 * N,), jnp.float32),
            pltpu.VMEM((3 * N,), jnp.float32),
            pltpu.VMEM((K,), jnp.int32),
            pltpu.VMEM((K,), jnp.int32),
            pltpu.VMEM((K,), jnp.int32),
            pltpu.VMEM((K,), jnp.int32),
        ],
    )
    return fn(us, va, vo, js, iaa, iao, zz)


def _loss_body(fl_ref, lab_ref, pred_ref, loss_ref):
    b = pl.program_id(0)
    f = fl_ref[0]
    lab = lab_ref[0]
    f0, f1, f2 = f[0:1, :], f[1:2, :], f[2:3, :]
    absum = jnp.abs(f0) + jnp.abs(f1) + jnp.abs(f2)
    valid = (absum > 0).astype(jnp.float32)
    mx = jnp.maximum(jnp.maximum(f0, f1), f2)
    se = jnp.exp(f0 - mx) + jnp.exp(f1 - mx) + jnp.exp(f2 - mx)
    lse = jnp.log(se) + mx
    flab = jnp.where(lab == 0, f0, jnp.where(lab == 1, f1, f2))
    nll = lse - flab
    wl = jnp.where(lab == 0, jnp.float32(1.0),
                   jnp.where(lab == 1, jnp.float32(2.0), jnp.float32(4.0)))
    wl = wl * valid
    num = jnp.sum(nll * wl, axis=(0, 1), keepdims=True)
    den = jnp.maximum(jnp.sum(wl, axis=(0, 1), keepdims=True),
                      jnp.float32(1e-6))
    lossb = num / den
    idx = jnp.zeros_like(lab)
    best = f0
    idx = jnp.where(f1 > best, 1, idx)
    best = jnp.maximum(best, f1)
    idx = jnp.where(f2 > best, 2, idx)
    pred_ref[0] = jnp.where(valid > 0, idx, -1)

    @pl.when(b == 0)
    def _():
        loss_ref[...] = lossb

    @pl.when(b > 0)
    def _():
        loss_ref[...] = loss_ref[...] + lossb


def _tc_loss(fl_t, lab3):
    return pl.pallas_call(
        _loss_body,
        grid=(B,),
        in_specs=[
            pl.BlockSpec((1, 3, N), lambda b: (b, 0, 0)),
            pl.BlockSpec((1, 1, N), lambda b: (b, 0, 0)),
        ],
        out_specs=[
            pl.BlockSpec((1, 1, N), lambda b: (b, 0, 0)),
            pl.BlockSpec((1, 1), lambda b: (0, 0)),
        ],
        out_shape=[
            jax.ShapeDtypeStruct((B, 1, N), jnp.int32),
            jax.ShapeDtypeStruct((1, 1), jnp.float32),
        ],
    )(fl_t, lab3)


def kernel(A2O_aspect_hidden_states, A2O_opinion_hidden_states,
           O2A_aspect_hidden_states, O2A_opinion_hidden_states,
           W_A2O, b_A2O, W_O2A, b_O2A,
           asp_idx_a2o, opi_idx_a2o, asp_idx_o2a, opi_idx_o2a,
           sentiment_labels):
    aa = A2O_aspect_hidden_states.astype(jnp.float32)
    ao = A2O_opinion_hidden_states.astype(jnp.float32)
    oa = O2A_aspect_hidden_states.astype(jnp.float32)
    oo = O2A_opinion_hidden_states.astype(jnp.float32)
    ia_a = asp_idx_a2o.astype(jnp.int32)
    ja_a = opi_idx_a2o.astype(jnp.int32)
    ia_o = asp_idx_o2a.astype(jnp.int32)
    ja_o = opi_idx_o2a.astype(jnp.int32)

    idx_a = jnp.stack([ia_a, ja_a], axis=1).reshape(2 * B * K)
    idx_o = jnp.stack([ia_o, ja_o], axis=1).reshape(2 * B * K)
    g_rows, usrc = _sc_gather(aa.reshape(B * N, H), oo.reshape(B * N, H),
                              oa.reshape(B * N, H), idx_a, idx_o, ia_o)
    g_rows = g_rows.reshape(4 * B, K, H)
    usrc = usrc.reshape(B, K, H)

    va, vo = _tc_v(ao, oo, W_A2O[H:], W_O2A[H:])

    aspc = jnp.stack([ia_a, ia_o], axis=1).reshape(2 * B, K, 1)
    opir = jnp.stack([ja_a, ja_o], axis=1).reshape(2 * B, 1, K)
    w1s = jnp.stack([W_A2O[:H], W_O2A[:H]]).astype(jnp.float32)
    b1s = jnp.stack([b_A2O.reshape(1, 3), b_O2A.reshape(1, 3)]).astype(jnp.float32)
    js, us = _tc_argmax(g_rows, usrc, w1s, b1s, aspc.astype(jnp.float32),
                        opir.astype(jnp.float32))
    js = js.reshape(2 * B, K)

    fl_rm, fl_t = _sc_combine(us.reshape(2 * B, 3 * K),
                              va.reshape(B, 3 * N), vo.reshape(B, 3 * N),
                              js, ia_a, ia_o,
                              jnp.zeros((3 * N,), jnp.float32))
    fl = fl_rm.reshape(B, N, 3)

    lab3 = sentiment_labels.astype(jnp.int32).reshape(B, 1, N)
    pred, loss = _tc_loss(fl_t.reshape(B, 3, N), lab3)
    return fl, pred.reshape(B, N), loss.reshape(())
